# Initial kernel scaffold; baseline (speedup 1.0000x reference)
import jax
import jax.numpy as jnp
from jax.experimental import pallas as pl


def kernel(pred, target):
    raise NotImplementedError("write your pallas kernel here")


# speedup vs baseline: 1.6233x; 1.6233x over previous
import functools
import math

import jax
import jax.numpy as jnp
import numpy as np
from jax.experimental import pallas as pl
from jax.experimental.pallas import tpu as pltpu

_SMOOTHING = 0.1
_BN = 64


def _loss_body(eps, coef_g, c_row, v, tgt_ref, pred_ref, out_ref):
    i = pl.program_id(0)
    t = tgt_ref[...]
    x = pred_ref[...]
    bn = x.shape[0]
    valid = t != 0
    col = jax.lax.broadcasted_iota(jnp.int32, x.shape, 1)
    colmask = (col > 0) & (col < v)
    xm = jnp.where(valid & colmask, x, 0.0)
    part = jnp.sum(xm)
    g = jnp.where(valid & (col == t), x, 0.0)
    gpart = jnp.sum(g)
    cnt = jnp.sum(jnp.where(valid, 1.0, 0.0))

    @pl.when(i == 0)
    def _():
        out_ref[0, 0] = 0.0

    out_ref[0, 0] += c_row * cnt - eps * part - coef_g * gpart


def kernel(pred, target):
    n, v = pred.shape
    eps = _SMOOTHING / (v - 2)
    eps32 = float(np.float32(eps))
    c_row = (v - 2) * (eps32 * math.log(eps32)) + 0.9 * math.log(0.9)
    coef_g = (1.0 - _SMOOTHING) - eps

    tgt2d = target.reshape(n, 1)
    grid = (n // _BN,)
    out = pl.pallas_call(
        functools.partial(_loss_body, eps, coef_g, c_row, v),
        grid=grid,
        in_specs=[
            pl.BlockSpec((_BN, 1), lambda i: (i, 0)),
            pl.BlockSpec((_BN, v), lambda i: (i, 0)),
        ],
        out_specs=pl.BlockSpec(
            (1, 1), lambda i: (0, 0), memory_space=pltpu.SMEM
        ),
        out_shape=jax.ShapeDtypeStruct((1, 1), jnp.float32),
    )(tgt2d, pred)
    return out[0, 0]
